# baseline (speedup 1.0000x reference)
import functools

import jax
import jax.numpy as jnp
from jax import lax
from jax.experimental import pallas as pl
from jax.experimental.pallas import tpu as pltpu
from jax.experimental.pallas import tpu_sc as plsc

N = 10000
T = 26667
CH = 32
TPAD = 28672
T4 = TPAD * 4
NROWS = 10112
SUBR = 632
LASTR = N - 15 * SUBR
DUMMY = N
SUB_T = TPAD // 16
SUB_CH = SUB_T // CH
W_CH = T4 // 128 // 32

RB = 1000
GRID = N // RB

_MESH = dict(core_axis_name="c", subcore_axis_name="s")


def _leaky(x):
    return jnp.where(x > 0, x, 0.01 * x)


def _sc_degree(sidx):
    @functools.partial(
        pl.kernel,
        out_type=jax.ShapeDtypeStruct((2, NROWS, 128), jnp.float32),
        mesh=plsc.VectorSubcoreMesh(**_MESH),
        scratch_types=[
            pltpu.VMEM((128,), jnp.int32),
            pltpu.VMEM((128, 128), jnp.float32),
            pltpu.VMEM_SHARED((NROWS, 128), jnp.float32),
        ],
    )
    def k(sidx_hbm, out_hbm, si_v, ones_v, scnt):
        c = lax.axis_index("c")
        s = lax.axis_index("s")
        w = c * 16 + s
        base = s * SUBR

        def fill_zero(i, carry):
            for g in range(8):
                ones_v[i, pl.ds(g * 16, 16)] = jnp.zeros((16,), jnp.float32)
            return carry

        lax.fori_loop(0, 128, fill_zero, 0)

        def zcopy(kk, carry):
            pltpu.sync_copy(ones_v, scnt.at[pl.ds(base + kk * 128, 128)])
            return carry

        lax.fori_loop(0, 4, zcopy, 0)
        pltpu.sync_copy(ones_v.at[pl.ds(0, 120)],
                        scnt.at[pl.ds(base + 512, 120)])

        def fill_ones(i, carry):
            for g in range(8):
                ones_v[i, pl.ds(g * 16, 16)] = jnp.ones((16,), jnp.float32)
            return carry

        lax.fori_loop(0, 128, fill_ones, 0)
        plsc.subcore_barrier()

        def chunk(kk, carry):
            soff = (w * W_CH + kk) * 128
            pltpu.sync_copy(sidx_hbm.at[pl.ds(soff, 128)], si_v)
            pltpu.sync_copy(ones_v, scnt.at[si_v], add=True)
            return carry

        lax.fori_loop(0, W_CH, chunk, 0)
        plsc.subcore_barrier()

        @pl.when(s < 15)
        def _out_full():
            pltpu.sync_copy(scnt.at[pl.ds(base, SUBR)],
                            out_hbm.at[c, pl.ds(base, SUBR)])

        @pl.when(s == 15)
        def _out_tail():
            pltpu.sync_copy(scnt.at[pl.ds(base, LASTR)],
                            out_hbm.at[c, pl.ds(base, LASTR)])

    return k(sidx)


def _sc_conv(gidx, sidx, hp_cat, colsplit):
    ng = 8

    nch = SUB_CH if colsplit else W_CH

    @functools.partial(
        pl.kernel,
        out_type=jax.ShapeDtypeStruct((2, N, 128), jnp.float32),
        mesh=plsc.VectorSubcoreMesh(**_MESH),
        scratch_types=[
            pltpu.VMEM((128,), jnp.int32),
            pltpu.VMEM((128,), jnp.int32),
            pltpu.VMEM((128, 128), jnp.float32),
            pltpu.VMEM((128, 128), jnp.float32),
            pltpu.VMEM_SHARED((NROWS, 128), jnp.float32),
            pltpu.SemaphoreType.DMA,
        ],
    )
    def k(gidx_hbm, sidx_hbm, hp_hbm, out_hbm, gi_v, si_v, rows_v, outs_v,
          shared, sem0):
        c = lax.axis_index("c")
        s = lax.axis_index("s")

        base = s * SUBR

        def cid_of(cur):
            if colsplit:
                return s * SUB_CH + cur
            return (c * 16 + s) * W_CH + cur

        def goff_of(cid):
            if colsplit:
                return c * T4 + cid * 128
            return cid * 128

        z0 = outs_v

        def zfill(i, carry):
            for g in range(ng):
                z0[i, pl.ds(g * 16, 16)] = jnp.zeros((16,), jnp.float32)
            return carry

        lax.fori_loop(0, 128, zfill, 0)

        def zcopy(kk, carry):
            pltpu.sync_copy(z0, shared.at[pl.ds(base + kk * 128, 128)])
            return carry

        lax.fori_loop(0, 4, zcopy, 0)
        pltpu.sync_copy(z0.at[pl.ds(0, 120)],
                        shared.at[pl.ds(base + 512, 120)])

        plsc.subcore_barrier()

        def chunk(kk, carry):
            cid = cid_of(kk)
            pltpu.sync_copy(gidx_hbm.at[pl.ds(goff_of(cid), 128)], gi_v)
            pltpu.sync_copy(sidx_hbm.at[pl.ds(cid * 128, 128)], si_v)
            pltpu.async_copy(hp_hbm.at[gi_v], rows_v, sem0).wait()

            def tet(jk, inner):
                for jj in range(2):
                    r = 8 * jk + 4 * jj
                    for g in range(ng):
                        sl = pl.ds(g * 16, 16)
                        r0 = rows_v[r, sl]
                        r1 = rows_v[r + 1, sl]
                        r2 = rows_v[r + 2, sl]
                        r3 = rows_v[r + 3, sl]
                        s4 = (r0 + r1) + (r2 + r3)
                        outs_v[r, sl] = s4 - r0
                        outs_v[r + 1, sl] = s4 - r1
                        outs_v[r + 2, sl] = s4 - r2
                        outs_v[r + 3, sl] = s4 - r3
                return inner

            lax.fori_loop(0, CH // 2, tet, 0)
            pltpu.sync_copy(outs_v, shared.at[si_v], add=True)
            return carry

        lax.fori_loop(0, nch, chunk, 0)
        plsc.subcore_barrier()

        @pl.when(s < 15)
        def _out_full():
            pltpu.sync_copy(shared.at[pl.ds(base, SUBR)],
                            out_hbm.at[c, pl.ds(base, SUBR)])

        @pl.when(s == 15)
        def _out_tail():
            pltpu.sync_copy(shared.at[pl.ds(base, LASTR)],
                            out_hbm.at[c, pl.ds(base, LASTR)])

    return k(gidx, sidx, hp_cat)


def _dis(cnt_ref):
    cnt = cnt_ref[0] + cnt_ref[1]
    return lax.rsqrt(1.0 + 3.0 * cnt)


def _tc1(cnt2, x_vert, W_in1, b_in1, W_g1a):
    def body(cnt_ref, xv_ref, w1_ref, b1_ref, wg_ref, x0_ref, hp_ref):
        dis = _dis(cnt_ref)
        x0 = jnp.dot(xv_ref[...], w1_ref[...],
                     preferred_element_type=jnp.float32) + b1_ref[...]
        t = jnp.dot(_leaky(x0), wg_ref[...],
                    preferred_element_type=jnp.float32)
        hp = dis * t
        x0_ref[...] = x0
        hp_ref[0] = hp[:, :128]
        hp_ref[1] = hp[:, 128:]

    return pl.pallas_call(
        body,
        grid=(GRID,),
        in_specs=[
            pl.BlockSpec((2, RB, 1), lambda i: (0, i, 0)),
            pl.BlockSpec((RB, 128), lambda i: (i, 0)),
            pl.BlockSpec((128, 256), lambda i: (0, 0)),
            pl.BlockSpec((1, 256), lambda i: (0, 0)),
            pl.BlockSpec((256, 256), lambda i: (0, 0)),
        ],
        out_specs=[
            pl.BlockSpec((RB, 256), lambda i: (i, 0)),
            pl.BlockSpec((2, RB, 128), lambda i: (0, i, 0)),
        ],
        out_shape=[
            jax.ShapeDtypeStruct((N, 256), jnp.float32),
            jax.ShapeDtypeStruct((2, N, 128), jnp.float32),
        ],
    )(cnt2, x_vert, W_in1, b_in1, W_g1a)


def _tc_mid(cnt2, agg, hp_prev, b_prev, W_next):
    g = W_next.shape[0]
    go = W_next.shape[1]
    hg = g // 2
    ho = go // 2

    def body(cnt_ref, agg_ref, hpp_ref, bp_ref, w_ref, hp_ref):
        dis = _dis(cnt_ref)
        h_lo = _leaky(dis * (agg_ref[0] + hpp_ref[0]) + bp_ref[:, :hg])
        h_hi = _leaky(dis * (agg_ref[1] + hpp_ref[1]) + bp_ref[:, hg:])
        t = (jnp.dot(h_lo, w_ref[:hg, :], preferred_element_type=jnp.float32)
             + jnp.dot(h_hi, w_ref[hg:, :],
                       preferred_element_type=jnp.float32))
        hp = dis * t
        hp_ref[0] = hp[:, :ho]
        hp_ref[1] = hp[:, ho:]

    return pl.pallas_call(
        body,
        grid=(GRID,),
        in_specs=[
            pl.BlockSpec((2, RB, 1), lambda i: (0, i, 0)),
            pl.BlockSpec((2, RB, hg), lambda i: (0, i, 0)),
            pl.BlockSpec((2, RB, hg), lambda i: (0, i, 0)),
            pl.BlockSpec((1, g), lambda i: (0, 0)),
            pl.BlockSpec((g, go), lambda i: (0, 0)),
        ],
        out_specs=[pl.BlockSpec((2, RB, ho), lambda i: (0, i, 0))],
        out_shape=[jax.ShapeDtypeStruct((2, N, ho), jnp.float32)],
    )(cnt2, agg, hp_prev, b_prev, W_next)[0]


def _tc_mid2(cnt2, agg, hp_prev, b_prev, W_next):
    def body(cnt_ref, agg_ref, hpp_ref, bp_ref, w_ref, hp_ref):
        dis = _dis(cnt_ref)
        h = _leaky(dis * (agg_ref[0] + agg_ref[1] + hpp_ref[...])
                   + bp_ref[...])
        t = jnp.dot(h, w_ref[...], preferred_element_type=jnp.float32)
        hp_ref[...] = dis * t

    return pl.pallas_call(
        body,
        grid=(GRID,),
        in_specs=[
            pl.BlockSpec((2, RB, 1), lambda i: (0, i, 0)),
            pl.BlockSpec((2, RB, 128), lambda i: (0, i, 0)),
            pl.BlockSpec((RB, 128), lambda i: (i, 0)),
            pl.BlockSpec((1, 128), lambda i: (0, 0)),
            pl.BlockSpec((128, 128), lambda i: (0, 0)),
        ],
        out_specs=[pl.BlockSpec((RB, 128), lambda i: (i, 0))],
        out_shape=[jax.ShapeDtypeStruct((N, 128), jnp.float32)],
    )(cnt2, agg, hp_prev, b_prev, W_next)[0]


def _tc3(cnt2, agg2, hp2, b_g1b, x0, W_in2, b_in2, W_g2a):
    def body(cnt_ref, agg_ref, hpp_ref, bp_ref, x0_ref, w2_ref, b2_ref,
             wg_ref, x2_ref, hp_ref):
        dis = _dis(cnt_ref)
        h_lo = dis * (agg_ref[0] + hpp_ref[0]) + bp_ref[:, :128]
        h_hi = dis * (agg_ref[1] + hpp_ref[1]) + bp_ref[:, 128:]
        inv = 1.0 / jnp.sqrt(jnp.float32(2.0))
        x1_lo = _leaky((x0_ref[:, :128] + h_lo) * inv)
        x1_hi = _leaky((x0_ref[:, 128:] + h_hi) * inv)
        x2 = (jnp.dot(x1_lo, w2_ref[:128, :],
                      preferred_element_type=jnp.float32)
              + jnp.dot(x1_hi, w2_ref[128:, :],
                        preferred_element_type=jnp.float32)) + b2_ref[...]
        t = jnp.dot(_leaky(x2), wg_ref[...],
                    preferred_element_type=jnp.float32)
        x2_ref[...] = x2
        hp_ref[...] = dis * t

    return pl.pallas_call(
        body,
        grid=(GRID,),
        in_specs=[
            pl.BlockSpec((2, RB, 1), lambda i: (0, i, 0)),
            pl.BlockSpec((2, RB, 128), lambda i: (0, i, 0)),
            pl.BlockSpec((2, RB, 128), lambda i: (0, i, 0)),
            pl.BlockSpec((1, 256), lambda i: (0, 0)),
            pl.BlockSpec((RB, 256), lambda i: (i, 0)),
            pl.BlockSpec((256, 128), lambda i: (0, 0)),
            pl.BlockSpec((1, 128), lambda i: (0, 0)),
            pl.BlockSpec((128, 128), lambda i: (0, 0)),
        ],
        out_specs=[
            pl.BlockSpec((RB, 128), lambda i: (i, 0)),
            pl.BlockSpec((RB, 128), lambda i: (i, 0)),
        ],
        out_shape=[
            jax.ShapeDtypeStruct((N, 128), jnp.float32),
            jax.ShapeDtypeStruct((N, 128), jnp.float32),
        ],
    )(cnt2, agg2, hp2, b_g1b, x0, W_in2, b_in2, W_g2a)


def _tc5(cnt2, agg4, hp4, b_g2b, x2, W_out, b_out, W_l1, b_l1, W_l2, b_l2):
    def body(cnt_ref, agg_ref, hpp_ref, bp_ref, x2_ref, wo_ref, bo_ref,
             w1_ref, b1_ref, w2_ref, b2_ref, out_ref):
        dis = _dis(cnt_ref)
        h4 = dis * (agg_ref[0] + agg_ref[1] + hpp_ref[...]) + bp_ref[...]
        inv = 1.0 / jnp.sqrt(jnp.float32(2.0))
        x3 = (x2_ref[...] + h4) * inv
        y = jnp.dot(x3, wo_ref[...],
                    preferred_element_type=jnp.float32) + bo_ref[...]
        y = jnp.dot(_leaky(y), w1_ref[...],
                    preferred_element_type=jnp.float32) + b1_ref[...]
        y = jnp.dot(_leaky(y), w2_ref[...],
                    preferred_element_type=jnp.float32) + b2_ref[...]
        out_ref[...] = y

    return pl.pallas_call(
        body,
        grid=(GRID,),
        in_specs=[
            pl.BlockSpec((2, RB, 1), lambda i: (0, i, 0)),
            pl.BlockSpec((2, RB, 128), lambda i: (0, i, 0)),
            pl.BlockSpec((RB, 128), lambda i: (i, 0)),
            pl.BlockSpec((1, 128), lambda i: (0, 0)),
            pl.BlockSpec((RB, 128), lambda i: (i, 0)),
            pl.BlockSpec((128, 128), lambda i: (0, 0)),
            pl.BlockSpec((1, 128), lambda i: (0, 0)),
            pl.BlockSpec((128, 64), lambda i: (0, 0)),
            pl.BlockSpec((1, 64), lambda i: (0, 0)),
            pl.BlockSpec((64, 8), lambda i: (0, 0)),
            pl.BlockSpec((1, 8), lambda i: (0, 0)),
        ],
        out_specs=[pl.BlockSpec((RB, 8), lambda i: (i, 0))],
        out_shape=[jax.ShapeDtypeStruct((N, 8), jnp.float32)],
    )(cnt2, agg4, hp4, b_g2b, x2, W_out, b_out, W_l1, b_l1, W_l2,
      b_l2)[0]


def kernel(x_vert, tets, W_in1, b_in1, W_g1a, b_g1a, W_g1b, b_g1b, W_in2,
           b_in2, W_g2a, b_g2a, W_g2b, b_g2b, W_out, b_out, W_l1, b_l1,
           W_l2, b_l2):
    tets_flat = tets.reshape(-1).astype(jnp.int32)
    npad = T4 - tets_flat.shape[0]
    sidx = jnp.concatenate(
        [tets_flat, jnp.full((npad,), DUMMY, jnp.int32)])
    g0 = jnp.concatenate([tets_flat, jnp.zeros((npad,), jnp.int32)])
    gidx2 = jnp.concatenate([g0, g0 + N])

    cnt_raw = _sc_degree(sidx)
    cnt2 = cnt_raw[:, :N, 0:1]

    b_in1r = b_in1.reshape(1, -1)
    x0, hp1 = _tc1(cnt2, x_vert, W_in1, b_in1r, W_g1a)

    agg1 = _sc_conv(gidx2, sidx, hp1.reshape(2 * N, 128), True)
    hp2 = _tc_mid(cnt2, agg1, hp1, b_g1a.reshape(1, -1), W_g1b)
    agg2 = _sc_conv(gidx2, sidx, hp2.reshape(2 * N, 128), True)

    x2, hp3 = _tc3(cnt2, agg2, hp2, b_g1b.reshape(1, -1), x0, W_in2,
                   b_in2.reshape(1, -1), W_g2a)
    agg3 = _sc_conv(g0, sidx, hp3, False)
    hp4 = _tc_mid2(cnt2, agg3, hp3, b_g2a.reshape(1, -1), W_g2b)
    agg4 = _sc_conv(g0, sidx, hp4, False)

    h8 = _tc5(cnt2, agg4, hp4, b_g2b.reshape(1, -1), x2, W_out,
              b_out.reshape(1, -1), W_l1, b_l1.reshape(1, -1), W_l2,
              b_l2.reshape(1, -1))
    return (h8[:, :3], h8[:, 3], h8[:, 4:])


# speedup vs baseline: 1.0003x; 1.0003x over previous
import functools

import jax
import jax.numpy as jnp
from jax import lax
from jax.experimental import pallas as pl
from jax.experimental.pallas import tpu as pltpu
from jax.experimental.pallas import tpu_sc as plsc

N = 10000
T = 26667
CH = 32
TPAD = 28672
T4 = TPAD * 4
NROWS = 10112
SUBR = 632
LASTR = N - 15 * SUBR
DUMMY = N
SUB_T = TPAD // 16
SUB_CH = SUB_T // CH
W_CH = T4 // 128 // 32

RB = 1000
GRID = N // RB

_MESH = dict(core_axis_name="c", subcore_axis_name="s")


def _leaky(x):
    return jnp.where(x > 0, x, 0.01 * x)


def _sc_degree(sidx):
    @functools.partial(
        pl.kernel,
        out_type=jax.ShapeDtypeStruct((2, NROWS, 128), jnp.float32),
        mesh=plsc.VectorSubcoreMesh(**_MESH),
        scratch_types=[
            pltpu.VMEM((128,), jnp.int32),
            pltpu.VMEM((128, 128), jnp.float32),
            pltpu.VMEM_SHARED((NROWS, 128), jnp.float32),
        ],
    )
    def k(sidx_hbm, out_hbm, si_v, ones_v, scnt):
        c = lax.axis_index("c")
        s = lax.axis_index("s")
        w = c * 16 + s
        base = s * SUBR

        def fill_zero(i, carry):
            for g in range(8):
                ones_v[i, pl.ds(g * 16, 16)] = jnp.zeros((16,), jnp.float32)
            return carry

        lax.fori_loop(0, 128, fill_zero, 0)

        def zcopy(kk, carry):
            pltpu.sync_copy(ones_v, scnt.at[pl.ds(base + kk * 128, 128)])
            return carry

        lax.fori_loop(0, 4, zcopy, 0)
        pltpu.sync_copy(ones_v.at[pl.ds(0, 120)],
                        scnt.at[pl.ds(base + 512, 120)])

        def fill_ones(i, carry):
            for g in range(8):
                ones_v[i, pl.ds(g * 16, 16)] = jnp.ones((16,), jnp.float32)
            return carry

        lax.fori_loop(0, 128, fill_ones, 0)
        plsc.subcore_barrier()

        def chunk(kk, carry):
            soff = (w * W_CH + kk) * 128
            pltpu.sync_copy(sidx_hbm.at[pl.ds(soff, 128)], si_v)
            pltpu.sync_copy(ones_v, scnt.at[si_v], add=True)
            return carry

        lax.fori_loop(0, W_CH, chunk, 0)
        plsc.subcore_barrier()

        @pl.when(s < 15)
        def _out_full():
            pltpu.sync_copy(scnt.at[pl.ds(base, SUBR)],
                            out_hbm.at[c, pl.ds(base, SUBR)])

        @pl.when(s == 15)
        def _out_tail():
            pltpu.sync_copy(scnt.at[pl.ds(base, LASTR)],
                            out_hbm.at[c, pl.ds(base, LASTR)])

    return k(sidx)


def _sc_conv(gidx, sidx, hp_cat, colsplit):
    ng = 8

    nch = SUB_CH if colsplit else W_CH

    @functools.partial(
        pl.kernel,
        out_type=jax.ShapeDtypeStruct((2, N, 128), jnp.float32),
        mesh=plsc.VectorSubcoreMesh(**_MESH),
        scratch_types=[
            pltpu.VMEM((128,), jnp.int32),
            pltpu.VMEM((128,), jnp.int32),
            pltpu.VMEM((128, 128), jnp.float32),
            pltpu.VMEM((128, 128), jnp.float32),
            pltpu.VMEM_SHARED((NROWS, 128), jnp.float32),
            pltpu.SemaphoreType.DMA,
        ],
    )
    def k(gidx_hbm, sidx_hbm, hp_hbm, out_hbm, gi_v, si_v, rows_v, outs_v,
          shared, sem0):
        c = lax.axis_index("c")
        s = lax.axis_index("s")

        base = s * SUBR

        def cid_of(cur):
            if colsplit:
                return s * SUB_CH + cur
            return (c * 16 + s) * W_CH + cur

        def goff_of(cid):
            if colsplit:
                return c * T4 + cid * 128
            return cid * 128

        z0 = outs_v

        def zfill(i, carry):
            for g in range(ng):
                z0[i, pl.ds(g * 16, 16)] = jnp.zeros((16,), jnp.float32)
            return carry

        lax.fori_loop(0, 128, zfill, 0)

        def zcopy(kk, carry):
            pltpu.sync_copy(z0, shared.at[pl.ds(base + kk * 128, 128)])
            return carry

        lax.fori_loop(0, 4, zcopy, 0)
        pltpu.sync_copy(z0.at[pl.ds(0, 120)],
                        shared.at[pl.ds(base + 512, 120)])

        plsc.subcore_barrier()

        def chunk(kk, carry):
            cid = cid_of(kk)
            pltpu.sync_copy(gidx_hbm.at[pl.ds(goff_of(cid), 128)], gi_v)
            pltpu.sync_copy(sidx_hbm.at[pl.ds(cid * 128, 128)], si_v)
            pltpu.async_copy(hp_hbm.at[gi_v], rows_v, sem0).wait()

            def tet(jk, inner):
                r = 4 * jk
                for g in range(ng):
                    sl = pl.ds(g * 16, 16)
                    r0 = rows_v[r, sl]
                    r1 = rows_v[r + 1, sl]
                    r2 = rows_v[r + 2, sl]
                    r3 = rows_v[r + 3, sl]
                    s4 = (r0 + r1) + (r2 + r3)
                    outs_v[r, sl] = s4 - r0
                    outs_v[r + 1, sl] = s4 - r1
                    outs_v[r + 2, sl] = s4 - r2
                    outs_v[r + 3, sl] = s4 - r3
                return inner

            lax.fori_loop(0, CH, tet, 0)
            pltpu.sync_copy(outs_v, shared.at[si_v], add=True)
            return carry

        lax.fori_loop(0, nch, chunk, 0)
        plsc.subcore_barrier()

        @pl.when(s < 15)
        def _out_full():
            pltpu.sync_copy(shared.at[pl.ds(base, SUBR)],
                            out_hbm.at[c, pl.ds(base, SUBR)])

        @pl.when(s == 15)
        def _out_tail():
            pltpu.sync_copy(shared.at[pl.ds(base, LASTR)],
                            out_hbm.at[c, pl.ds(base, LASTR)])

    return k(gidx, sidx, hp_cat)


def _dis(cnt_ref):
    cnt = cnt_ref[0] + cnt_ref[1]
    return lax.rsqrt(1.0 + 3.0 * cnt)


def _tc1(cnt2, x_vert, W_in1, b_in1, W_g1a):
    def body(cnt_ref, xv_ref, w1_ref, b1_ref, wg_ref, x0_ref, hp_ref):
        dis = _dis(cnt_ref)
        x0 = jnp.dot(xv_ref[...], w1_ref[...],
                     preferred_element_type=jnp.float32) + b1_ref[...]
        t = jnp.dot(_leaky(x0), wg_ref[...],
                    preferred_element_type=jnp.float32)
        hp = dis * t
        x0_ref[...] = x0
        hp_ref[0] = hp[:, :128]
        hp_ref[1] = hp[:, 128:]

    return pl.pallas_call(
        body,
        grid=(GRID,),
        in_specs=[
            pl.BlockSpec((2, RB, 1), lambda i: (0, i, 0)),
            pl.BlockSpec((RB, 128), lambda i: (i, 0)),
            pl.BlockSpec((128, 256), lambda i: (0, 0)),
            pl.BlockSpec((1, 256), lambda i: (0, 0)),
            pl.BlockSpec((256, 256), lambda i: (0, 0)),
        ],
        out_specs=[
            pl.BlockSpec((RB, 256), lambda i: (i, 0)),
            pl.BlockSpec((2, RB, 128), lambda i: (0, i, 0)),
        ],
        out_shape=[
            jax.ShapeDtypeStruct((N, 256), jnp.float32),
            jax.ShapeDtypeStruct((2, N, 128), jnp.float32),
        ],
    )(cnt2, x_vert, W_in1, b_in1, W_g1a)


def _tc_mid(cnt2, agg, hp_prev, b_prev, W_next):
    g = W_next.shape[0]
    go = W_next.shape[1]
    hg = g // 2
    ho = go // 2

    def body(cnt_ref, agg_ref, hpp_ref, bp_ref, w_ref, hp_ref):
        dis = _dis(cnt_ref)
        h_lo = _leaky(dis * (agg_ref[0] + hpp_ref[0]) + bp_ref[:, :hg])
        h_hi = _leaky(dis * (agg_ref[1] + hpp_ref[1]) + bp_ref[:, hg:])
        t = (jnp.dot(h_lo, w_ref[:hg, :], preferred_element_type=jnp.float32)
             + jnp.dot(h_hi, w_ref[hg:, :],
                       preferred_element_type=jnp.float32))
        hp = dis * t
        hp_ref[0] = hp[:, :ho]
        hp_ref[1] = hp[:, ho:]

    return pl.pallas_call(
        body,
        grid=(GRID,),
        in_specs=[
            pl.BlockSpec((2, RB, 1), lambda i: (0, i, 0)),
            pl.BlockSpec((2, RB, hg), lambda i: (0, i, 0)),
            pl.BlockSpec((2, RB, hg), lambda i: (0, i, 0)),
            pl.BlockSpec((1, g), lambda i: (0, 0)),
            pl.BlockSpec((g, go), lambda i: (0, 0)),
        ],
        out_specs=[pl.BlockSpec((2, RB, ho), lambda i: (0, i, 0))],
        out_shape=[jax.ShapeDtypeStruct((2, N, ho), jnp.float32)],
    )(cnt2, agg, hp_prev, b_prev, W_next)[0]


def _tc_mid2(cnt2, agg, hp_prev, b_prev, W_next):
    def body(cnt_ref, agg_ref, hpp_ref, bp_ref, w_ref, hp_ref):
        dis = _dis(cnt_ref)
        h = _leaky(dis * (agg_ref[0] + agg_ref[1] + hpp_ref[...])
                   + bp_ref[...])
        t = jnp.dot(h, w_ref[...], preferred_element_type=jnp.float32)
        hp_ref[...] = dis * t

    return pl.pallas_call(
        body,
        grid=(GRID,),
        in_specs=[
            pl.BlockSpec((2, RB, 1), lambda i: (0, i, 0)),
            pl.BlockSpec((2, RB, 128), lambda i: (0, i, 0)),
            pl.BlockSpec((RB, 128), lambda i: (i, 0)),
            pl.BlockSpec((1, 128), lambda i: (0, 0)),
            pl.BlockSpec((128, 128), lambda i: (0, 0)),
        ],
        out_specs=[pl.BlockSpec((RB, 128), lambda i: (i, 0))],
        out_shape=[jax.ShapeDtypeStruct((N, 128), jnp.float32)],
    )(cnt2, agg, hp_prev, b_prev, W_next)[0]


def _tc3(cnt2, agg2, hp2, b_g1b, x0, W_in2, b_in2, W_g2a):
    def body(cnt_ref, agg_ref, hpp_ref, bp_ref, x0_ref, w2_ref, b2_ref,
             wg_ref, x2_ref, hp_ref):
        dis = _dis(cnt_ref)
        h_lo = dis * (agg_ref[0] + hpp_ref[0]) + bp_ref[:, :128]
        h_hi = dis * (agg_ref[1] + hpp_ref[1]) + bp_ref[:, 128:]
        inv = 1.0 / jnp.sqrt(jnp.float32(2.0))
        x1_lo = _leaky((x0_ref[:, :128] + h_lo) * inv)
        x1_hi = _leaky((x0_ref[:, 128:] + h_hi) * inv)
        x2 = (jnp.dot(x1_lo, w2_ref[:128, :],
                      preferred_element_type=jnp.float32)
              + jnp.dot(x1_hi, w2_ref[128:, :],
                        preferred_element_type=jnp.float32)) + b2_ref[...]
        t = jnp.dot(_leaky(x2), wg_ref[...],
                    preferred_element_type=jnp.float32)
        x2_ref[...] = x2
        hp_ref[...] = dis * t

    return pl.pallas_call(
        body,
        grid=(GRID,),
        in_specs=[
            pl.BlockSpec((2, RB, 1), lambda i: (0, i, 0)),
            pl.BlockSpec((2, RB, 128), lambda i: (0, i, 0)),
            pl.BlockSpec((2, RB, 128), lambda i: (0, i, 0)),
            pl.BlockSpec((1, 256), lambda i: (0, 0)),
            pl.BlockSpec((RB, 256), lambda i: (i, 0)),
            pl.BlockSpec((256, 128), lambda i: (0, 0)),
            pl.BlockSpec((1, 128), lambda i: (0, 0)),
            pl.BlockSpec((128, 128), lambda i: (0, 0)),
        ],
        out_specs=[
            pl.BlockSpec((RB, 128), lambda i: (i, 0)),
            pl.BlockSpec((RB, 128), lambda i: (i, 0)),
        ],
        out_shape=[
            jax.ShapeDtypeStruct((N, 128), jnp.float32),
            jax.ShapeDtypeStruct((N, 128), jnp.float32),
        ],
    )(cnt2, agg2, hp2, b_g1b, x0, W_in2, b_in2, W_g2a)


def _tc5(cnt2, agg4, hp4, b_g2b, x2, W_out, b_out, W_l1, b_l1, W_l2, b_l2):
    def body(cnt_ref, agg_ref, hpp_ref, bp_ref, x2_ref, wo_ref, bo_ref,
             w1_ref, b1_ref, w2_ref, b2_ref, out_ref):
        dis = _dis(cnt_ref)
        h4 = dis * (agg_ref[0] + agg_ref[1] + hpp_ref[...]) + bp_ref[...]
        inv = 1.0 / jnp.sqrt(jnp.float32(2.0))
        x3 = (x2_ref[...] + h4) * inv
        y = jnp.dot(x3, wo_ref[...],
                    preferred_element_type=jnp.float32) + bo_ref[...]
        y = jnp.dot(_leaky(y), w1_ref[...],
                    preferred_element_type=jnp.float32) + b1_ref[...]
        y = jnp.dot(_leaky(y), w2_ref[...],
                    preferred_element_type=jnp.float32) + b2_ref[...]
        out_ref[...] = y

    return pl.pallas_call(
        body,
        grid=(GRID,),
        in_specs=[
            pl.BlockSpec((2, RB, 1), lambda i: (0, i, 0)),
            pl.BlockSpec((2, RB, 128), lambda i: (0, i, 0)),
            pl.BlockSpec((RB, 128), lambda i: (i, 0)),
            pl.BlockSpec((1, 128), lambda i: (0, 0)),
            pl.BlockSpec((RB, 128), lambda i: (i, 0)),
            pl.BlockSpec((128, 128), lambda i: (0, 0)),
            pl.BlockSpec((1, 128), lambda i: (0, 0)),
            pl.BlockSpec((128, 64), lambda i: (0, 0)),
            pl.BlockSpec((1, 64), lambda i: (0, 0)),
            pl.BlockSpec((64, 8), lambda i: (0, 0)),
            pl.BlockSpec((1, 8), lambda i: (0, 0)),
        ],
        out_specs=[pl.BlockSpec((RB, 8), lambda i: (i, 0))],
        out_shape=[jax.ShapeDtypeStruct((N, 8), jnp.float32)],
    )(cnt2, agg4, hp4, b_g2b, x2, W_out, b_out, W_l1, b_l1, W_l2,
      b_l2)[0]


def kernel(x_vert, tets, W_in1, b_in1, W_g1a, b_g1a, W_g1b, b_g1b, W_in2,
           b_in2, W_g2a, b_g2a, W_g2b, b_g2b, W_out, b_out, W_l1, b_l1,
           W_l2, b_l2):
    tets_flat = tets.reshape(-1).astype(jnp.int32)
    npad = T4 - tets_flat.shape[0]
    sidx = jnp.concatenate(
        [tets_flat, jnp.full((npad,), DUMMY, jnp.int32)])
    g0 = jnp.concatenate([tets_flat, jnp.zeros((npad,), jnp.int32)])
    gidx2 = jnp.concatenate([g0, g0 + N])

    cnt_raw = _sc_degree(sidx)
    cnt2 = cnt_raw[:, :N, 0:1]

    b_in1r = b_in1.reshape(1, -1)
    x0, hp1 = _tc1(cnt2, x_vert, W_in1, b_in1r, W_g1a)

    agg1 = _sc_conv(gidx2, sidx, hp1.reshape(2 * N, 128), True)
    hp2 = _tc_mid(cnt2, agg1, hp1, b_g1a.reshape(1, -1), W_g1b)
    agg2 = _sc_conv(gidx2, sidx, hp2.reshape(2 * N, 128), True)

    x2, hp3 = _tc3(cnt2, agg2, hp2, b_g1b.reshape(1, -1), x0, W_in2,
                   b_in2.reshape(1, -1), W_g2a)
    agg3 = _sc_conv(g0, sidx, hp3, False)
    hp4 = _tc_mid2(cnt2, agg3, hp3, b_g2a.reshape(1, -1), W_g2b)
    agg4 = _sc_conv(g0, sidx, hp4, False)

    h8 = _tc5(cnt2, agg4, hp4, b_g2b.reshape(1, -1), x2, W_out,
              b_out.reshape(1, -1), W_l1, b_l1.reshape(1, -1), W_l2,
              b_l2.reshape(1, -1))
    return (h8[:, :3], h8[:, 3], h8[:, 4:])


# speedup vs baseline: 1.3953x; 1.3949x over previous
import functools

import jax
import jax.numpy as jnp
from jax import lax
from jax.experimental import pallas as pl
from jax.experimental.pallas import tpu as pltpu
from jax.experimental.pallas import tpu_sc as plsc

N = 10000
T = 26667
CH = 32
TPAD = 27648
T4 = TPAD * 4
NROWS = 10112
SUBR = 632
LASTR = N - 15 * SUBR
DUMMY = N
SUB_T = TPAD // 16
SUB_CH = SUB_T // CH
W_CH = T4 // 128 // 32

RB = 1000
GRID = N // RB

_MESH = dict(core_axis_name="c", subcore_axis_name="s")


def _leaky(x):
    return jnp.where(x > 0, x, 0.01 * x)


def _sc_degree(sidx):
    @functools.partial(
        pl.kernel,
        out_type=jax.ShapeDtypeStruct((2, NROWS, 128), jnp.float32),
        mesh=plsc.VectorSubcoreMesh(**_MESH),
        scratch_types=[
            pltpu.VMEM((128,), jnp.int32),
            pltpu.VMEM((128, 128), jnp.float32),
            pltpu.VMEM_SHARED((NROWS, 128), jnp.float32),
        ],
    )
    def k(sidx_hbm, out_hbm, si_v, ones_v, scnt):
        c = lax.axis_index("c")
        s = lax.axis_index("s")
        w = c * 16 + s
        base = s * SUBR

        def fill_zero(i, carry):
            for g in range(8):
                ones_v[i, pl.ds(g * 16, 16)] = jnp.zeros((16,), jnp.float32)
            return carry

        lax.fori_loop(0, 128, fill_zero, 0)

        def zcopy(kk, carry):
            pltpu.sync_copy(ones_v, scnt.at[pl.ds(base + kk * 128, 128)])
            return carry

        lax.fori_loop(0, 4, zcopy, 0)
        pltpu.sync_copy(ones_v.at[pl.ds(0, 120)],
                        scnt.at[pl.ds(base + 512, 120)])

        def fill_ones(i, carry):
            for g in range(8):
                ones_v[i, pl.ds(g * 16, 16)] = jnp.ones((16,), jnp.float32)
            return carry

        lax.fori_loop(0, 128, fill_ones, 0)
        plsc.subcore_barrier()

        def chunk(kk, carry):
            soff = (w * W_CH + kk) * 128
            pltpu.sync_copy(sidx_hbm.at[pl.ds(soff, 128)], si_v)
            pltpu.sync_copy(ones_v, scnt.at[si_v], add=True)
            return carry

        lax.fori_loop(0, W_CH, chunk, 0)
        plsc.subcore_barrier()

        @pl.when(s < 15)
        def _out_full():
            pltpu.sync_copy(scnt.at[pl.ds(base, SUBR)],
                            out_hbm.at[c, pl.ds(base, SUBR)])

        @pl.when(s == 15)
        def _out_tail():
            pltpu.sync_copy(scnt.at[pl.ds(base, LASTR)],
                            out_hbm.at[c, pl.ds(base, LASTR)])

    return k(sidx)


def _sc_conv(gidx, sidx, hp_cat, colsplit):
    ng = 8

    nch = SUB_CH if colsplit else W_CH

    @functools.partial(
        pl.kernel,
        out_type=jax.ShapeDtypeStruct((2, N, 128), jnp.float32),
        mesh=plsc.VectorSubcoreMesh(**_MESH),
        scratch_types=[
            pltpu.VMEM((128,), jnp.int32),
            pltpu.VMEM((128,), jnp.int32),
            pltpu.VMEM((128, 128), jnp.float32),
            pltpu.VMEM((128, 128), jnp.float32),
            pltpu.VMEM_SHARED((NROWS, 128), jnp.float32),
            pltpu.SemaphoreType.DMA,
        ],
    )
    def k(gidx_hbm, sidx_hbm, hp_hbm, out_hbm, gi_v, si_v, rows_v, outs_v,
          shared, sem0):
        c = lax.axis_index("c")
        s = lax.axis_index("s")

        base = s * SUBR

        def cid_of(cur):
            if colsplit:
                return s * SUB_CH + cur
            return (c * 16 + s) * W_CH + cur

        def goff_of(cid):
            if colsplit:
                return c * T4 + cid * 128
            return cid * 128

        z0 = outs_v

        def zfill(i, carry):
            for g in range(ng):
                z0[i, pl.ds(g * 16, 16)] = jnp.zeros((16,), jnp.float32)
            return carry

        lax.fori_loop(0, 128, zfill, 0)

        def zcopy(kk, carry):
            pltpu.sync_copy(z0, shared.at[pl.ds(base + kk * 128, 128)])
            return carry

        lax.fori_loop(0, 4, zcopy, 0)
        pltpu.sync_copy(z0.at[pl.ds(0, 120)],
                        shared.at[pl.ds(base + 512, 120)])

        plsc.subcore_barrier()

        def chunk(kk, carry):
            cid = cid_of(kk)
            pltpu.sync_copy(gidx_hbm.at[pl.ds(goff_of(cid), 128)], gi_v)
            pltpu.sync_copy(sidx_hbm.at[pl.ds(cid * 128, 128)], si_v)
            pltpu.async_copy(hp_hbm.at[gi_v], rows_v, sem0).wait()

            def tet(jk, inner):
                r = 4 * jk
                for g in range(ng):
                    sl = pl.ds(g * 16, 16)
                    r0 = rows_v[r, sl]
                    r1 = rows_v[r + 1, sl]
                    r2 = rows_v[r + 2, sl]
                    r3 = rows_v[r + 3, sl]
                    s4 = (r0 + r1) + (r2 + r3)
                    outs_v[r, sl] = s4 - r0
                    outs_v[r + 1, sl] = s4 - r1
                    outs_v[r + 2, sl] = s4 - r2
                    outs_v[r + 3, sl] = s4 - r3
                return inner

            lax.fori_loop(0, CH, tet, 0)
            pltpu.sync_copy(outs_v, shared.at[si_v], add=True)
            return carry

        lax.fori_loop(0, nch, chunk, 0)
        plsc.subcore_barrier()

        @pl.when(s < 15)
        def _out_full():
            pltpu.sync_copy(shared.at[pl.ds(base, SUBR)],
                            out_hbm.at[c, pl.ds(base, SUBR)])

        @pl.when(s == 15)
        def _out_tail():
            pltpu.sync_copy(shared.at[pl.ds(base, LASTR)],
                            out_hbm.at[c, pl.ds(base, LASTR)])

    return k(gidx, sidx, hp_cat)


def _dis(cnt_ref):
    cnt = cnt_ref[0] + cnt_ref[1]
    return lax.rsqrt(1.0 + 3.0 * cnt)


def _tc1(cnt2, x_vert, W_in1, b_in1, W_g1a):
    def body(cnt_ref, xv_ref, w1_ref, b1_ref, wg_ref, x0_ref, hp_ref):
        dis = _dis(cnt_ref)
        x0 = jnp.dot(xv_ref[...], w1_ref[...],
                     preferred_element_type=jnp.float32) + b1_ref[...]
        t = jnp.dot(_leaky(x0), wg_ref[...],
                    preferred_element_type=jnp.float32)
        hp = dis * t
        x0_ref[...] = x0
        hp_ref[0] = hp[:, :128]
        hp_ref[1] = hp[:, 128:]

    return pl.pallas_call(
        body,
        grid=(GRID,),
        in_specs=[
            pl.BlockSpec((2, RB, 1), lambda i: (0, i, 0)),
            pl.BlockSpec((RB, 128), lambda i: (i, 0)),
            pl.BlockSpec((128, 256), lambda i: (0, 0)),
            pl.BlockSpec((1, 256), lambda i: (0, 0)),
            pl.BlockSpec((256, 256), lambda i: (0, 0)),
        ],
        out_specs=[
            pl.BlockSpec((RB, 256), lambda i: (i, 0)),
            pl.BlockSpec((2, RB, 128), lambda i: (0, i, 0)),
        ],
        out_shape=[
            jax.ShapeDtypeStruct((N, 256), jnp.float32),
            jax.ShapeDtypeStruct((2, N, 128), jnp.float32),
        ],
    )(cnt2, x_vert, W_in1, b_in1, W_g1a)


def _tc_mid(cnt2, agg, hp_prev, b_prev, W_next):
    g = W_next.shape[0]
    go = W_next.shape[1]
    hg = g // 2
    ho = go // 2

    def body(cnt_ref, agg_ref, hpp_ref, bp_ref, w_ref, hp_ref):
        dis = _dis(cnt_ref)
        h_lo = _leaky(dis * (agg_ref[0] + hpp_ref[0]) + bp_ref[:, :hg])
        h_hi = _leaky(dis * (agg_ref[1] + hpp_ref[1]) + bp_ref[:, hg:])
        t = (jnp.dot(h_lo, w_ref[:hg, :], preferred_element_type=jnp.float32)
             + jnp.dot(h_hi, w_ref[hg:, :],
                       preferred_element_type=jnp.float32))
        hp = dis * t
        hp_ref[0] = hp[:, :ho]
        hp_ref[1] = hp[:, ho:]

    return pl.pallas_call(
        body,
        grid=(GRID,),
        in_specs=[
            pl.BlockSpec((2, RB, 1), lambda i: (0, i, 0)),
            pl.BlockSpec((2, RB, hg), lambda i: (0, i, 0)),
            pl.BlockSpec((2, RB, hg), lambda i: (0, i, 0)),
            pl.BlockSpec((1, g), lambda i: (0, 0)),
            pl.BlockSpec((g, go), lambda i: (0, 0)),
        ],
        out_specs=[pl.BlockSpec((2, RB, ho), lambda i: (0, i, 0))],
        out_shape=[jax.ShapeDtypeStruct((2, N, ho), jnp.float32)],
    )(cnt2, agg, hp_prev, b_prev, W_next)[0]


def _tc_mid2(cnt2, agg, hp_prev, b_prev, W_next):
    def body(cnt_ref, agg_ref, hpp_ref, bp_ref, w_ref, hp_ref):
        dis = _dis(cnt_ref)
        h = _leaky(dis * (agg_ref[0] + agg_ref[1] + hpp_ref[...])
                   + bp_ref[...])
        t = jnp.dot(h, w_ref[...], preferred_element_type=jnp.float32)
        hp_ref[...] = dis * t

    return pl.pallas_call(
        body,
        grid=(GRID,),
        in_specs=[
            pl.BlockSpec((2, RB, 1), lambda i: (0, i, 0)),
            pl.BlockSpec((2, RB, 128), lambda i: (0, i, 0)),
            pl.BlockSpec((RB, 128), lambda i: (i, 0)),
            pl.BlockSpec((1, 128), lambda i: (0, 0)),
            pl.BlockSpec((128, 128), lambda i: (0, 0)),
        ],
        out_specs=[pl.BlockSpec((RB, 128), lambda i: (i, 0))],
        out_shape=[jax.ShapeDtypeStruct((N, 128), jnp.float32)],
    )(cnt2, agg, hp_prev, b_prev, W_next)[0]


def _tc3(cnt2, agg2, hp2, b_g1b, x0, W_in2, b_in2, W_g2a):
    def body(cnt_ref, agg_ref, hpp_ref, bp_ref, x0_ref, w2_ref, b2_ref,
             wg_ref, x2_ref, hp_ref):
        dis = _dis(cnt_ref)
        h_lo = dis * (agg_ref[0] + hpp_ref[0]) + bp_ref[:, :128]
        h_hi = dis * (agg_ref[1] + hpp_ref[1]) + bp_ref[:, 128:]
        inv = 1.0 / jnp.sqrt(jnp.float32(2.0))
        x1_lo = _leaky((x0_ref[:, :128] + h_lo) * inv)
        x1_hi = _leaky((x0_ref[:, 128:] + h_hi) * inv)
        x2 = (jnp.dot(x1_lo, w2_ref[:128, :],
                      preferred_element_type=jnp.float32)
              + jnp.dot(x1_hi, w2_ref[128:, :],
                        preferred_element_type=jnp.float32)) + b2_ref[...]
        t = jnp.dot(_leaky(x2), wg_ref[...],
                    preferred_element_type=jnp.float32)
        x2_ref[...] = x2
        hp_ref[...] = dis * t

    return pl.pallas_call(
        body,
        grid=(GRID,),
        in_specs=[
            pl.BlockSpec((2, RB, 1), lambda i: (0, i, 0)),
            pl.BlockSpec((2, RB, 128), lambda i: (0, i, 0)),
            pl.BlockSpec((2, RB, 128), lambda i: (0, i, 0)),
            pl.BlockSpec((1, 256), lambda i: (0, 0)),
            pl.BlockSpec((RB, 256), lambda i: (i, 0)),
            pl.BlockSpec((256, 128), lambda i: (0, 0)),
            pl.BlockSpec((1, 128), lambda i: (0, 0)),
            pl.BlockSpec((128, 128), lambda i: (0, 0)),
        ],
        out_specs=[
            pl.BlockSpec((RB, 128), lambda i: (i, 0)),
            pl.BlockSpec((RB, 128), lambda i: (i, 0)),
        ],
        out_shape=[
            jax.ShapeDtypeStruct((N, 128), jnp.float32),
            jax.ShapeDtypeStruct((N, 128), jnp.float32),
        ],
    )(cnt2, agg2, hp2, b_g1b, x0, W_in2, b_in2, W_g2a)


def _tc5(cnt2, agg4, hp4, b_g2b, x2, W_out, b_out, W_l1, b_l1, W_l2, b_l2):
    def body(cnt_ref, agg_ref, hpp_ref, bp_ref, x2_ref, wo_ref, bo_ref,
             w1_ref, b1_ref, w2_ref, b2_ref, out_ref):
        dis = _dis(cnt_ref)
        h4 = dis * (agg_ref[0] + agg_ref[1] + hpp_ref[...]) + bp_ref[...]
        inv = 1.0 / jnp.sqrt(jnp.float32(2.0))
        x3 = (x2_ref[...] + h4) * inv
        y = jnp.dot(x3, wo_ref[...],
                    preferred_element_type=jnp.float32) + bo_ref[...]
        y = jnp.dot(_leaky(y), w1_ref[...],
                    preferred_element_type=jnp.float32) + b1_ref[...]
        y = jnp.dot(_leaky(y), w2_ref[...],
                    preferred_element_type=jnp.float32) + b2_ref[...]
        out_ref[...] = y

    return pl.pallas_call(
        body,
        grid=(GRID,),
        in_specs=[
            pl.BlockSpec((2, RB, 1), lambda i: (0, i, 0)),
            pl.BlockSpec((2, RB, 128), lambda i: (0, i, 0)),
            pl.BlockSpec((RB, 128), lambda i: (i, 0)),
            pl.BlockSpec((1, 128), lambda i: (0, 0)),
            pl.BlockSpec((RB, 128), lambda i: (i, 0)),
            pl.BlockSpec((128, 128), lambda i: (0, 0)),
            pl.BlockSpec((1, 128), lambda i: (0, 0)),
            pl.BlockSpec((128, 64), lambda i: (0, 0)),
            pl.BlockSpec((1, 64), lambda i: (0, 0)),
            pl.BlockSpec((64, 8), lambda i: (0, 0)),
            pl.BlockSpec((1, 8), lambda i: (0, 0)),
        ],
        out_specs=[pl.BlockSpec((RB, 8), lambda i: (i, 0))],
        out_shape=[jax.ShapeDtypeStruct((N, 8), jnp.float32)],
    )(cnt2, agg4, hp4, b_g2b, x2, W_out, b_out, W_l1, b_l1, W_l2,
      b_l2)[0]


def kernel(x_vert, tets, W_in1, b_in1, W_g1a, b_g1a, W_g1b, b_g1b, W_in2,
           b_in2, W_g2a, b_g2a, W_g2b, b_g2b, W_out, b_out, W_l1, b_l1,
           W_l2, b_l2):
    tets_flat = tets.reshape(-1).astype(jnp.int32)
    npad = T4 - tets_flat.shape[0]
    pad_rows = DUMMY + jnp.arange(npad, dtype=jnp.int32) % (NROWS - N)
    sidx = jnp.concatenate([tets_flat, pad_rows])
    g0 = jnp.concatenate([tets_flat, jnp.zeros((npad,), jnp.int32)])
    gidx2 = jnp.concatenate([g0, g0 + N])

    cnt_raw = _sc_degree(sidx)
    cnt2 = cnt_raw[:, :N, 0:1]

    b_in1r = b_in1.reshape(1, -1)
    x0, hp1 = _tc1(cnt2, x_vert, W_in1, b_in1r, W_g1a)

    agg1 = _sc_conv(gidx2, sidx, hp1.reshape(2 * N, 128), True)
    hp2 = _tc_mid(cnt2, agg1, hp1, b_g1a.reshape(1, -1), W_g1b)
    agg2 = _sc_conv(gidx2, sidx, hp2.reshape(2 * N, 128), True)

    x2, hp3 = _tc3(cnt2, agg2, hp2, b_g1b.reshape(1, -1), x0, W_in2,
                   b_in2.reshape(1, -1), W_g2a)
    agg3 = _sc_conv(g0, sidx, hp3, False)
    hp4 = _tc_mid2(cnt2, agg3, hp3, b_g2a.reshape(1, -1), W_g2b)
    agg4 = _sc_conv(g0, sidx, hp4, False)

    h8 = _tc5(cnt2, agg4, hp4, b_g2b.reshape(1, -1), x2, W_out,
              b_out.reshape(1, -1), W_l1, b_l1.reshape(1, -1), W_l2,
              b_l2.reshape(1, -1))
    return (h8[:, :3], h8[:, 3], h8[:, 4:])


# speedup vs baseline: 1.4940x; 1.0707x over previous
import functools

import jax
import jax.numpy as jnp
from jax import lax
from jax.experimental import pallas as pl
from jax.experimental.pallas import tpu as pltpu
from jax.experimental.pallas import tpu_sc as plsc

N = 10000
T = 26667
CH = 32
TPAD = 27648
T4 = TPAD * 4
NROWS = 10112
SUBR = 632
LASTR = N - 15 * SUBR
DUMMY = N
SUB_T = TPAD // 16
SUB_CH = SUB_T // CH
W_CH = T4 // 128 // 32

RB = 1000
GRID = N // RB

_MESH = dict(core_axis_name="c", subcore_axis_name="s")


def _leaky(x):
    return jnp.where(x > 0, x, 0.01 * x)


def _sc_degree(sidx):
    @functools.partial(
        pl.kernel,
        out_type=jax.ShapeDtypeStruct((2, NROWS, 128), jnp.float32),
        mesh=plsc.VectorSubcoreMesh(**_MESH),
        scratch_types=[
            pltpu.VMEM((128,), jnp.int32),
            pltpu.VMEM((128, 128), jnp.float32),
            pltpu.VMEM_SHARED((NROWS, 128), jnp.float32),
        ],
    )
    def k(sidx_hbm, out_hbm, si_v, ones_v, scnt):
        c = lax.axis_index("c")
        s = lax.axis_index("s")
        w = c * 16 + s
        base = s * SUBR

        def fill_zero(i, carry):
            for g in range(8):
                ones_v[i, pl.ds(g * 16, 16)] = jnp.zeros((16,), jnp.float32)
            return carry

        lax.fori_loop(0, 128, fill_zero, 0)

        def zcopy(kk, carry):
            pltpu.sync_copy(ones_v, scnt.at[pl.ds(base + kk * 128, 128)])
            return carry

        lax.fori_loop(0, 4, zcopy, 0)
        pltpu.sync_copy(ones_v.at[pl.ds(0, 120)],
                        scnt.at[pl.ds(base + 512, 120)])

        def fill_ones(i, carry):
            for g in range(8):
                ones_v[i, pl.ds(g * 16, 16)] = jnp.ones((16,), jnp.float32)
            return carry

        lax.fori_loop(0, 128, fill_ones, 0)
        plsc.subcore_barrier()

        def chunk(kk, carry):
            soff = (w * W_CH + kk) * 128
            pltpu.sync_copy(sidx_hbm.at[pl.ds(soff, 128)], si_v)
            pltpu.sync_copy(ones_v, scnt.at[si_v], add=True)
            return carry

        lax.fori_loop(0, W_CH, chunk, 0)
        plsc.subcore_barrier()

        @pl.when(s < 15)
        def _out_full():
            pltpu.sync_copy(scnt.at[pl.ds(base, SUBR)],
                            out_hbm.at[c, pl.ds(base, SUBR)])

        @pl.when(s == 15)
        def _out_tail():
            pltpu.sync_copy(scnt.at[pl.ds(base, LASTR)],
                            out_hbm.at[c, pl.ds(base, LASTR)])

    return k(sidx)


def _sc_conv(sidx, hp_cat, colsplit):
    ng = 8

    nch = SUB_CH if colsplit else W_CH

    @functools.partial(
        pl.kernel,
        out_type=jax.ShapeDtypeStruct((2, N, 128), jnp.float32),
        mesh=plsc.VectorSubcoreMesh(**_MESH),
        scratch_types=[
            pltpu.VMEM((128,), jnp.int32),
            pltpu.VMEM((128,), jnp.int32),
            pltpu.VMEM((128, 128), jnp.float32),
            pltpu.VMEM((128, 128), jnp.float32),
            pltpu.VMEM_SHARED((NROWS, 128), jnp.float32),
            pltpu.SemaphoreType.DMA,
        ],
    )
    def k(sidx_hbm, hp_hbm, out_hbm, gi_v, si_v, rows_v, outs_v,
          shared, sem0):
        c = lax.axis_index("c")
        s = lax.axis_index("s")

        base = s * SUBR
        coff = c * N if colsplit else 0

        def cid_of(cur):
            if colsplit:
                return s * SUB_CH + cur
            return (c * 16 + s) * W_CH + cur

        z0 = outs_v

        def zfill(i, carry):
            for g in range(ng):
                z0[i, pl.ds(g * 16, 16)] = jnp.zeros((16,), jnp.float32)
            return carry

        lax.fori_loop(0, 128, zfill, 0)

        def zcopy(kk, carry):
            pltpu.sync_copy(z0, shared.at[pl.ds(base + kk * 128, 128)])
            return carry

        lax.fori_loop(0, 4, zcopy, 0)
        pltpu.sync_copy(z0.at[pl.ds(0, 120)],
                        shared.at[pl.ds(base + 512, 120)])

        plsc.subcore_barrier()

        def chunk(kk, carry):
            cid = cid_of(kk)
            pltpu.sync_copy(sidx_hbm.at[pl.ds(cid * 128, 128)], si_v)
            for gk in range(8):
                sl = pl.ds(gk * 16, 16)
                si = si_v[sl]
                gi_v[sl] = jnp.where(si < N, si, 0) + coff
            pltpu.async_copy(hp_hbm.at[gi_v], rows_v, sem0).wait()

            def tet(jk, inner):
                r = 4 * jk
                for g in range(ng):
                    sl = pl.ds(g * 16, 16)
                    r0 = rows_v[r, sl]
                    r1 = rows_v[r + 1, sl]
                    r2 = rows_v[r + 2, sl]
                    r3 = rows_v[r + 3, sl]
                    s4 = (r0 + r1) + (r2 + r3)
                    outs_v[r, sl] = s4 - r0
                    outs_v[r + 1, sl] = s4 - r1
                    outs_v[r + 2, sl] = s4 - r2
                    outs_v[r + 3, sl] = s4 - r3
                return inner

            lax.fori_loop(0, CH, tet, 0)
            pltpu.sync_copy(outs_v, shared.at[si_v], add=True)
            return carry

        lax.fori_loop(0, nch, chunk, 0)
        plsc.subcore_barrier()

        @pl.when(s < 15)
        def _out_full():
            pltpu.sync_copy(shared.at[pl.ds(base, SUBR)],
                            out_hbm.at[c, pl.ds(base, SUBR)])

        @pl.when(s == 15)
        def _out_tail():
            pltpu.sync_copy(shared.at[pl.ds(base, LASTR)],
                            out_hbm.at[c, pl.ds(base, LASTR)])

    return k(sidx, hp_cat)


def _dis(cnt_ref):
    cnt = cnt_ref[0] + cnt_ref[1]
    return lax.rsqrt(1.0 + 3.0 * cnt)


def _tc1(cnt2, x_vert, W_in1, b_in1, W_g1a):
    def body(cnt_ref, xv_ref, w1_ref, b1_ref, wg_ref, x0_ref, hp_ref):
        dis = _dis(cnt_ref)
        x0 = jnp.dot(xv_ref[...], w1_ref[...],
                     preferred_element_type=jnp.float32) + b1_ref[...]
        t = jnp.dot(_leaky(x0), wg_ref[...],
                    preferred_element_type=jnp.float32)
        hp = dis * t
        x0_ref[...] = x0
        hp_ref[0] = hp[:, :128]
        hp_ref[1] = hp[:, 128:]

    return pl.pallas_call(
        body,
        grid=(GRID,),
        in_specs=[
            pl.BlockSpec((2, RB, 1), lambda i: (0, i, 0)),
            pl.BlockSpec((RB, 128), lambda i: (i, 0)),
            pl.BlockSpec((128, 256), lambda i: (0, 0)),
            pl.BlockSpec((1, 256), lambda i: (0, 0)),
            pl.BlockSpec((256, 256), lambda i: (0, 0)),
        ],
        out_specs=[
            pl.BlockSpec((RB, 256), lambda i: (i, 0)),
            pl.BlockSpec((2, RB, 128), lambda i: (0, i, 0)),
        ],
        out_shape=[
            jax.ShapeDtypeStruct((N, 256), jnp.float32),
            jax.ShapeDtypeStruct((2, N, 128), jnp.float32),
        ],
    )(cnt2, x_vert, W_in1, b_in1, W_g1a)


def _tc_mid(cnt2, agg, hp_prev, b_prev, W_next):
    g = W_next.shape[0]
    go = W_next.shape[1]
    hg = g // 2
    ho = go // 2

    def body(cnt_ref, agg_ref, hpp_ref, bp_ref, w_ref, hp_ref):
        dis = _dis(cnt_ref)
        h_lo = _leaky(dis * (agg_ref[0] + hpp_ref[0]) + bp_ref[:, :hg])
        h_hi = _leaky(dis * (agg_ref[1] + hpp_ref[1]) + bp_ref[:, hg:])
        t = (jnp.dot(h_lo, w_ref[:hg, :], preferred_element_type=jnp.float32)
             + jnp.dot(h_hi, w_ref[hg:, :],
                       preferred_element_type=jnp.float32))
        hp = dis * t
        hp_ref[0] = hp[:, :ho]
        hp_ref[1] = hp[:, ho:]

    return pl.pallas_call(
        body,
        grid=(GRID,),
        in_specs=[
            pl.BlockSpec((2, RB, 1), lambda i: (0, i, 0)),
            pl.BlockSpec((2, RB, hg), lambda i: (0, i, 0)),
            pl.BlockSpec((2, RB, hg), lambda i: (0, i, 0)),
            pl.BlockSpec((1, g), lambda i: (0, 0)),
            pl.BlockSpec((g, go), lambda i: (0, 0)),
        ],
        out_specs=[pl.BlockSpec((2, RB, ho), lambda i: (0, i, 0))],
        out_shape=[jax.ShapeDtypeStruct((2, N, ho), jnp.float32)],
    )(cnt2, agg, hp_prev, b_prev, W_next)[0]


def _tc_mid2(cnt2, agg, hp_prev, b_prev, W_next):
    def body(cnt_ref, agg_ref, hpp_ref, bp_ref, w_ref, hp_ref):
        dis = _dis(cnt_ref)
        h = _leaky(dis * (agg_ref[0] + agg_ref[1] + hpp_ref[...])
                   + bp_ref[...])
        t = jnp.dot(h, w_ref[...], preferred_element_type=jnp.float32)
        hp_ref[...] = dis * t

    return pl.pallas_call(
        body,
        grid=(GRID,),
        in_specs=[
            pl.BlockSpec((2, RB, 1), lambda i: (0, i, 0)),
            pl.BlockSpec((2, RB, 128), lambda i: (0, i, 0)),
            pl.BlockSpec((RB, 128), lambda i: (i, 0)),
            pl.BlockSpec((1, 128), lambda i: (0, 0)),
            pl.BlockSpec((128, 128), lambda i: (0, 0)),
        ],
        out_specs=[pl.BlockSpec((RB, 128), lambda i: (i, 0))],
        out_shape=[jax.ShapeDtypeStruct((N, 128), jnp.float32)],
    )(cnt2, agg, hp_prev, b_prev, W_next)[0]


def _tc3(cnt2, agg2, hp2, b_g1b, x0, W_in2, b_in2, W_g2a):
    def body(cnt_ref, agg_ref, hpp_ref, bp_ref, x0_ref, w2_ref, b2_ref,
             wg_ref, x2_ref, hp_ref):
        dis = _dis(cnt_ref)
        h_lo = dis * (agg_ref[0] + hpp_ref[0]) + bp_ref[:, :128]
        h_hi = dis * (agg_ref[1] + hpp_ref[1]) + bp_ref[:, 128:]
        inv = 1.0 / jnp.sqrt(jnp.float32(2.0))
        x1_lo = _leaky((x0_ref[:, :128] + h_lo) * inv)
        x1_hi = _leaky((x0_ref[:, 128:] + h_hi) * inv)
        x2 = (jnp.dot(x1_lo, w2_ref[:128, :],
                      preferred_element_type=jnp.float32)
              + jnp.dot(x1_hi, w2_ref[128:, :],
                        preferred_element_type=jnp.float32)) + b2_ref[...]
        t = jnp.dot(_leaky(x2), wg_ref[...],
                    preferred_element_type=jnp.float32)
        x2_ref[...] = x2
        hp_ref[...] = dis * t

    return pl.pallas_call(
        body,
        grid=(GRID,),
        in_specs=[
            pl.BlockSpec((2, RB, 1), lambda i: (0, i, 0)),
            pl.BlockSpec((2, RB, 128), lambda i: (0, i, 0)),
            pl.BlockSpec((2, RB, 128), lambda i: (0, i, 0)),
            pl.BlockSpec((1, 256), lambda i: (0, 0)),
            pl.BlockSpec((RB, 256), lambda i: (i, 0)),
            pl.BlockSpec((256, 128), lambda i: (0, 0)),
            pl.BlockSpec((1, 128), lambda i: (0, 0)),
            pl.BlockSpec((128, 128), lambda i: (0, 0)),
        ],
        out_specs=[
            pl.BlockSpec((RB, 128), lambda i: (i, 0)),
            pl.BlockSpec((RB, 128), lambda i: (i, 0)),
        ],
        out_shape=[
            jax.ShapeDtypeStruct((N, 128), jnp.float32),
            jax.ShapeDtypeStruct((N, 128), jnp.float32),
        ],
    )(cnt2, agg2, hp2, b_g1b, x0, W_in2, b_in2, W_g2a)


def _tc5(cnt2, agg4, hp4, b_g2b, x2, W_out, b_out, W_l1, b_l1, W_l2, b_l2):
    def body(cnt_ref, agg_ref, hpp_ref, bp_ref, x2_ref, wo_ref, bo_ref,
             w1_ref, b1_ref, w2_ref, b2_ref, out_ref):
        dis = _dis(cnt_ref)
        h4 = dis * (agg_ref[0] + agg_ref[1] + hpp_ref[...]) + bp_ref[...]
        inv = 1.0 / jnp.sqrt(jnp.float32(2.0))
        x3 = (x2_ref[...] + h4) * inv
        y = jnp.dot(x3, wo_ref[...],
                    preferred_element_type=jnp.float32) + bo_ref[...]
        y = jnp.dot(_leaky(y), w1_ref[...],
                    preferred_element_type=jnp.float32) + b1_ref[...]
        y = jnp.dot(_leaky(y), w2_ref[...],
                    preferred_element_type=jnp.float32) + b2_ref[...]
        out_ref[...] = y

    return pl.pallas_call(
        body,
        grid=(GRID,),
        in_specs=[
            pl.BlockSpec((2, RB, 1), lambda i: (0, i, 0)),
            pl.BlockSpec((2, RB, 128), lambda i: (0, i, 0)),
            pl.BlockSpec((RB, 128), lambda i: (i, 0)),
            pl.BlockSpec((1, 128), lambda i: (0, 0)),
            pl.BlockSpec((RB, 128), lambda i: (i, 0)),
            pl.BlockSpec((128, 128), lambda i: (0, 0)),
            pl.BlockSpec((1, 128), lambda i: (0, 0)),
            pl.BlockSpec((128, 64), lambda i: (0, 0)),
            pl.BlockSpec((1, 64), lambda i: (0, 0)),
            pl.BlockSpec((64, 8), lambda i: (0, 0)),
            pl.BlockSpec((1, 8), lambda i: (0, 0)),
        ],
        out_specs=[pl.BlockSpec((RB, 8), lambda i: (i, 0))],
        out_shape=[jax.ShapeDtypeStruct((N, 8), jnp.float32)],
    )(cnt2, agg4, hp4, b_g2b, x2, W_out, b_out, W_l1, b_l1, W_l2,
      b_l2)[0]


def kernel(x_vert, tets, W_in1, b_in1, W_g1a, b_g1a, W_g1b, b_g1b, W_in2,
           b_in2, W_g2a, b_g2a, W_g2b, b_g2b, W_out, b_out, W_l1, b_l1,
           W_l2, b_l2):
    tets_flat = tets.reshape(-1).astype(jnp.int32)
    npad = T4 - tets_flat.shape[0]
    pad_rows = DUMMY + jnp.arange(npad, dtype=jnp.int32) % (NROWS - N)
    sidx = jnp.concatenate([tets_flat, pad_rows])
    cnt_raw = _sc_degree(sidx)
    cnt2 = cnt_raw[:, :N, 0:1]

    b_in1r = b_in1.reshape(1, -1)
    x0, hp1 = _tc1(cnt2, x_vert, W_in1, b_in1r, W_g1a)

    agg1 = _sc_conv(sidx, hp1.reshape(2 * N, 128), True)
    hp2 = _tc_mid(cnt2, agg1, hp1, b_g1a.reshape(1, -1), W_g1b)
    agg2 = _sc_conv(sidx, hp2.reshape(2 * N, 128), True)

    x2, hp3 = _tc3(cnt2, agg2, hp2, b_g1b.reshape(1, -1), x0, W_in2,
                   b_in2.reshape(1, -1), W_g2a)
    agg3 = _sc_conv(sidx, hp3, False)
    hp4 = _tc_mid2(cnt2, agg3, hp3, b_g2a.reshape(1, -1), W_g2b)
    agg4 = _sc_conv(sidx, hp4, False)

    h8 = _tc5(cnt2, agg4, hp4, b_g2b.reshape(1, -1), x2, W_out,
              b_out.reshape(1, -1), W_l1, b_l1.reshape(1, -1), W_l2,
              b_l2.reshape(1, -1))
    return (h8[:, :3], h8[:, 3], h8[:, 4:])
